# baseline (speedup 1.0000x reference)
import jax
import jax.numpy as jnp
from jax.experimental import pallas as pl
from jax.experimental.pallas import tpu as pltpu

_E = 8


def _pool_body(x_ref, wg_ref, logits_ref):
    xb = x_ref[0]
    xg = jnp.mean(xb, axis=0, keepdims=True)
    lg = jnp.dot(xg, wg_ref[...],
                 preferred_element_type=jnp.float32)
    logits_ref[...] = jnp.clip(lg, -50.0, 50.0)[None]


def _route_body(logits_ref, eidx_ref, gvals_ref, loss_ref):
    l = logits_ref[...][:, 0, :]
    B = l.shape[0]
    iota = jax.lax.broadcasted_iota(jnp.int32, (B, _E), 1)
    v0 = jnp.max(l, axis=1, keepdims=True)
    e0 = jnp.min(jnp.where(l == v0, iota, _E), axis=1, keepdims=True)
    masked = jnp.where(iota == e0, -jnp.inf, l)
    v1 = jnp.max(masked, axis=1, keepdims=True)
    e1 = jnp.min(jnp.where(masked == v1, iota, _E), axis=1, keepdims=True)
    t = jnp.exp(v1 - v0)
    g0 = 1.0 / (1.0 + t)
    g1 = t / (1.0 + t)
    eidx_ref[...] = jnp.concatenate([e0, e1], axis=1)
    gvals_ref[...] = jnp.concatenate([g0, g1], axis=1)

    gates = (jnp.where(iota == e0, g0, 0.0)
             + jnp.where(iota == e1, g1, 0.0))
    n = float(_E)
    eps = 1e-10

    def cv_sq(v):
        m = jnp.sum(v) / n
        var = jnp.sum((v - m) ** 2) / (n - 1.0)
        return var / (m * m + eps)

    imp = jnp.sum(gates, axis=0, keepdims=True)
    load = jnp.sum((gates > 0.0).astype(jnp.float32), axis=0, keepdims=True)
    loss = cv_sq(imp) + cv_sq(load)
    loss_ref[...] = jnp.clip(loss, 0.0, 1000.0) * jnp.ones((1, 1), jnp.float32)


def _expert_body(eidx_ref, gvals_ref, x_ref, W1_ref, b1_ref, W2_ref, b2_ref,
                 y_ref):
    b = pl.program_id(0)
    xb = x_ref[0]
    e0 = eidx_ref[b, 0]
    e1 = eidx_ref[b, 1]
    g0 = gvals_ref[b, 0]
    g1 = gvals_ref[b, 1]

    def expert(e):
        h = jnp.dot(xb, W1_ref[e], preferred_element_type=jnp.float32)
        h = h + b1_ref[e][None, :]
        h = 0.5 * h * (1.0 + jax.lax.erf(h * 0.7071067811865476))
        o = jnp.dot(h, W2_ref[e], preferred_element_type=jnp.float32)
        return o + b2_ref[e][None, :]

    y_ref[0] = g0 * expert(e0) + g1 * expert(e1)


def kernel(x, w_gate, W1, b1, W2, b2):
    B, H, W, C = x.shape
    T = H * W
    E = w_gate.shape[1]
    x_flat = x.reshape(B, T, C)

    logits = pl.pallas_call(
        _pool_body,
        grid=(B,),
        in_specs=[
            pl.BlockSpec((1, T, C), lambda b: (b, 0, 0)),
            pl.BlockSpec((C, E), lambda b: (0, 0)),
        ],
        out_specs=pl.BlockSpec((1, 1, E), lambda b: (b, 0, 0)),
        out_shape=jax.ShapeDtypeStruct((B, 1, E), jnp.float32),
        compiler_params=pltpu.CompilerParams(
            dimension_semantics=("parallel",),
        ),
    )(x_flat, w_gate)

    eidx, gvals, loss = pl.pallas_call(
        _route_body,
        out_shape=[
            jax.ShapeDtypeStruct((B, 2), jnp.int32),
            jax.ShapeDtypeStruct((B, 2), jnp.float32),
            jax.ShapeDtypeStruct((1, 1), jnp.float32),
        ],
    )(logits)

    y_flat = pl.pallas_call(
        _expert_body,
        grid=(B,),
        in_specs=[
            pl.BlockSpec(memory_space=pltpu.SMEM),
            pl.BlockSpec(memory_space=pltpu.SMEM),
            pl.BlockSpec((1, T, C), lambda b: (b, 0, 0)),
            pl.BlockSpec(W1.shape, lambda b: (0, 0, 0)),
            pl.BlockSpec(b1.shape, lambda b: (0, 0)),
            pl.BlockSpec(W2.shape, lambda b: (0, 0, 0)),
            pl.BlockSpec(b2.shape, lambda b: (0, 0)),
        ],
        out_specs=pl.BlockSpec((1, T, C), lambda b: (b, 0, 0)),
        out_shape=jax.ShapeDtypeStruct((B, T, C), jnp.float32),
        compiler_params=pltpu.CompilerParams(
            dimension_semantics=("parallel",),
        ),
    )(eidx, gvals, x_flat, W1, b1, W2, b2)

    return y_flat.reshape(B, H, W, C), loss[0, 0]


# speedup vs baseline: 1.2005x; 1.2005x over previous
import jax
import jax.numpy as jnp
from jax.experimental import pallas as pl
from jax.experimental.pallas import tpu as pltpu

_E = 8


def _moe_body(x_ref, wg_ref, W1_ref, b1_ref, W2_ref, b2_ref,
              y_ref, gates_ref):
    xb = x_ref[0]
    xg = jnp.mean(xb, axis=0, keepdims=True)
    logits = jnp.dot(xg, wg_ref[...],
                     preferred_element_type=jnp.float32)
    logits = jnp.clip(logits, -50.0, 50.0)
    iota = jax.lax.broadcasted_iota(jnp.int32, (1, _E), 1)
    v0 = jnp.max(logits)
    e0 = jnp.min(jnp.where(logits == v0, iota, _E))
    masked = jnp.where(iota == e0, -jnp.inf, logits)
    v1 = jnp.max(masked)
    e1 = jnp.min(jnp.where(masked == v1, iota, _E))
    t = jnp.exp(v1 - v0)
    g0 = 1.0 / (1.0 + t)
    g1 = t / (1.0 + t)

    gates_ref[...] = (jnp.where(iota == e0, g0, 0.0)
                      + jnp.where(iota == e1, g1, 0.0))[None]

    xb16 = xb.astype(jnp.bfloat16)

    def expert(e):
        h = jnp.dot(xb16, W1_ref[e].astype(jnp.bfloat16),
                    preferred_element_type=jnp.float32)
        h = h + b1_ref[e][None, :]
        h = 0.5 * h * (1.0 + jax.lax.erf(h * 0.7071067811865476))
        o = jnp.dot(h.astype(jnp.bfloat16), W2_ref[e].astype(jnp.bfloat16),
                    preferred_element_type=jnp.float32)
        return o + b2_ref[e][None, :]

    y_ref[0] = g0 * expert(e0) + g1 * expert(e1)


def _loss_body(gates_ref, loss_ref):
    g = gates_ref[...][:, 0, :]
    n = float(_E)
    eps = 1e-10

    def cv_sq(v):
        m = jnp.sum(v) / n
        var = jnp.sum((v - m) ** 2) / (n - 1.0)
        return var / (m * m + eps)

    imp = jnp.sum(g, axis=0, keepdims=True)
    load = jnp.sum((g > 0.0).astype(jnp.float32), axis=0, keepdims=True)
    loss = cv_sq(imp) + cv_sq(load)
    loss_ref[...] = jnp.clip(loss, 0.0, 1000.0) * jnp.ones((1, 1), jnp.float32)


def kernel(x, w_gate, W1, b1, W2, b2):
    B, H, W, C = x.shape
    T = H * W
    E = w_gate.shape[1]
    x_flat = x.reshape(B, T, C)

    y_flat, gates = pl.pallas_call(
        _moe_body,
        grid=(B,),
        in_specs=[
            pl.BlockSpec((1, T, C), lambda b: (b, 0, 0)),
            pl.BlockSpec((C, E), lambda b: (0, 0)),
            pl.BlockSpec(W1.shape, lambda b: (0, 0, 0)),
            pl.BlockSpec(b1.shape, lambda b: (0, 0)),
            pl.BlockSpec(W2.shape, lambda b: (0, 0, 0)),
            pl.BlockSpec(b2.shape, lambda b: (0, 0)),
        ],
        out_specs=[
            pl.BlockSpec((1, T, C), lambda b: (b, 0, 0)),
            pl.BlockSpec((1, 1, E), lambda b: (b, 0, 0)),
        ],
        out_shape=[
            jax.ShapeDtypeStruct((B, T, C), jnp.float32),
            jax.ShapeDtypeStruct((B, 1, E), jnp.float32),
        ],
        compiler_params=pltpu.CompilerParams(
            dimension_semantics=("parallel",),
        ),
    )(x_flat, w_gate, W1, b1, W2, b2)

    loss = pl.pallas_call(
        _loss_body,
        out_shape=jax.ShapeDtypeStruct((1, 1), jnp.float32),
    )(gates)

    return y_flat.reshape(B, H, W, C), loss[0, 0]


# speedup vs baseline: 1.7897x; 1.4907x over previous
import functools

import jax
import jax.numpy as jnp
from jax.experimental import pallas as pl
from jax.experimental.pallas import tpu as pltpu

_E = 8
_G = 4


def _moe_body(x_ref, wg_ref, W1_ref, b1_ref, W2_ref, b2_ref,
              y_ref, gates_ref):
    xg = jnp.mean(x_ref[...], axis=1)
    logits = jnp.dot(xg, wg_ref[...],
                     preferred_element_type=jnp.float32)
    logits = jnp.clip(logits, -50.0, 50.0)
    iota = jax.lax.broadcasted_iota(jnp.int32, (_G, _E), 1)
    v0 = jnp.max(logits, axis=1, keepdims=True)
    e0 = jnp.min(jnp.where(logits == v0, iota, _E), axis=1, keepdims=True)
    masked = jnp.where(iota == e0, -jnp.inf, logits)
    v1 = jnp.max(masked, axis=1, keepdims=True)
    e1 = jnp.min(jnp.where(masked == v1, iota, _E), axis=1, keepdims=True)
    t = jnp.exp(v1 - v0)
    g0 = 1.0 / (1.0 + t)
    g1 = t / (1.0 + t)

    gates = (jnp.where(iota == e0, g0, 0.0)
             + jnp.where(iota == e1, g1, 0.0))
    gates_ref[...] = gates[:, None, :]

    def expert(xi, e):
        h = jnp.dot(xi, W1_ref[e], preferred_element_type=jnp.float32)
        h = h + b1_ref[e][None, :]
        h = 0.5 * h * (1.0 + jax.lax.erf(h * 0.7071067811865476))
        o = jnp.dot(h, W2_ref[e], preferred_element_type=jnp.float32)
        return o + b2_ref[e][None, :]

    for i in range(_G):
        li = logits[i:i + 1]
        ii = iota[i:i + 1]
        v0i = jnp.max(li)
        e0i = jnp.min(jnp.where(li == v0i, ii, _E))
        mi = jnp.where(ii == e0i, -jnp.inf, li)
        v1i = jnp.max(mi)
        e1i = jnp.min(jnp.where(mi == v1i, ii, _E))
        ti = jnp.exp(v1i - v0i)
        g0i = 1.0 / (1.0 + ti)
        g1i = ti / (1.0 + ti)
        xi = x_ref[i]
        y_ref[i] = g0i * expert(xi, e0i) + g1i * expert(xi, e1i)


def _loss_body(gates_ref, loss_ref):
    g = gates_ref[...][:, 0, :]
    n = float(_E)
    eps = 1e-10

    def cv_sq(v):
        m = jnp.sum(v) / n
        var = jnp.sum((v - m) ** 2) / (n - 1.0)
        return var / (m * m + eps)

    imp = jnp.sum(g, axis=0, keepdims=True)
    load = jnp.sum((g > 0.0).astype(jnp.float32), axis=0, keepdims=True)
    loss = cv_sq(imp) + cv_sq(load)
    loss_ref[...] = jnp.clip(loss, 0.0, 1000.0) * jnp.ones((1, 1), jnp.float32)


def kernel(x, w_gate, W1, b1, W2, b2):
    B, H, W, C = x.shape
    T = H * W
    E = w_gate.shape[1]
    x_flat = x.reshape(B, T, C)

    y_flat, gates = pl.pallas_call(
        _moe_body,
        grid=(B // _G,),
        in_specs=[
            pl.BlockSpec((_G, T, C), lambda b: (b, 0, 0)),
            pl.BlockSpec((C, E), lambda b: (0, 0)),
            pl.BlockSpec(W1.shape, lambda b: (0, 0, 0)),
            pl.BlockSpec(b1.shape, lambda b: (0, 0)),
            pl.BlockSpec(W2.shape, lambda b: (0, 0, 0)),
            pl.BlockSpec(b2.shape, lambda b: (0, 0)),
        ],
        out_specs=[
            pl.BlockSpec((_G, T, C), lambda b: (b, 0, 0)),
            pl.BlockSpec((_G, 1, E), lambda b: (b, 0, 0)),
        ],
        out_shape=[
            jax.ShapeDtypeStruct((B, T, C), jnp.float32),
            jax.ShapeDtypeStruct((B, 1, E), jnp.float32),
        ],
        compiler_params=pltpu.CompilerParams(
            dimension_semantics=("parallel",),
        ),
    )(x_flat, w_gate, W1, b1, W2, b2)

    loss = pl.pallas_call(
        _loss_body,
        out_shape=jax.ShapeDtypeStruct((1, 1), jnp.float32),
    )(gates)

    return y_flat.reshape(B, H, W, C), loss[0, 0]


# speedup vs baseline: 1.8956x; 1.0592x over previous
import functools

import jax
import jax.numpy as jnp
from jax.experimental import pallas as pl
from jax.experimental.pallas import tpu as pltpu

_E = 8
_G = 8


def _moe_body(x_ref, wg_ref, W1_ref, b1_ref, W2_ref, b2_ref,
              y_ref, gates_ref):
    xg = jnp.mean(x_ref[...], axis=1)
    logits = jnp.dot(xg, wg_ref[...],
                     preferred_element_type=jnp.float32)
    logits = jnp.clip(logits, -50.0, 50.0)
    iota = jax.lax.broadcasted_iota(jnp.int32, (_G, _E), 1)
    v0 = jnp.max(logits, axis=1, keepdims=True)
    e0 = jnp.min(jnp.where(logits == v0, iota, _E), axis=1, keepdims=True)
    masked = jnp.where(iota == e0, -jnp.inf, logits)
    v1 = jnp.max(masked, axis=1, keepdims=True)
    e1 = jnp.min(jnp.where(masked == v1, iota, _E), axis=1, keepdims=True)
    t = jnp.exp(v1 - v0)
    g0 = 1.0 / (1.0 + t)
    g1 = t / (1.0 + t)

    gates = (jnp.where(iota == e0, g0, 0.0)
             + jnp.where(iota == e1, g1, 0.0))
    gates_ref[...] = gates[:, None, :]

    def expert(xi, e):
        h = jnp.dot(xi, W1_ref[e], preferred_element_type=jnp.float32)
        h = h + b1_ref[e][None, :]
        h = 0.5 * h * (1.0 + jax.lax.erf(h * 0.7071067811865476))
        o = jnp.dot(h, W2_ref[e], preferred_element_type=jnp.float32)
        return o + b2_ref[e][None, :]

    for i in range(_G):
        li = logits[i:i + 1]
        ii = iota[i:i + 1]
        v0i = jnp.max(li)
        e0i = jnp.min(jnp.where(li == v0i, ii, _E))
        mi = jnp.where(ii == e0i, -jnp.inf, li)
        v1i = jnp.max(mi)
        e1i = jnp.min(jnp.where(mi == v1i, ii, _E))
        ti = jnp.exp(v1i - v0i)
        g0i = 1.0 / (1.0 + ti)
        g1i = ti / (1.0 + ti)
        xi = x_ref[i]
        y_ref[i] = g0i * expert(xi, e0i) + g1i * expert(xi, e1i)


def _loss_body(gates_ref, loss_ref):
    g = gates_ref[...][:, 0, :]
    n = float(_E)
    eps = 1e-10

    def cv_sq(v):
        m = jnp.sum(v) / n
        var = jnp.sum((v - m) ** 2) / (n - 1.0)
        return var / (m * m + eps)

    imp = jnp.sum(g, axis=0, keepdims=True)
    load = jnp.sum((g > 0.0).astype(jnp.float32), axis=0, keepdims=True)
    loss = cv_sq(imp) + cv_sq(load)
    loss_ref[...] = jnp.clip(loss, 0.0, 1000.0) * jnp.ones((1, 1), jnp.float32)


def kernel(x, w_gate, W1, b1, W2, b2):
    B, H, W, C = x.shape
    T = H * W
    E = w_gate.shape[1]
    x_flat = x.reshape(B, T, C)

    y_flat, gates = pl.pallas_call(
        _moe_body,
        grid=(B // _G,),
        in_specs=[
            pl.BlockSpec((_G, T, C), lambda b: (b, 0, 0)),
            pl.BlockSpec((C, E), lambda b: (0, 0)),
            pl.BlockSpec(W1.shape, lambda b: (0, 0, 0)),
            pl.BlockSpec(b1.shape, lambda b: (0, 0)),
            pl.BlockSpec(W2.shape, lambda b: (0, 0, 0)),
            pl.BlockSpec(b2.shape, lambda b: (0, 0)),
        ],
        out_specs=[
            pl.BlockSpec((_G, T, C), lambda b: (b, 0, 0)),
            pl.BlockSpec((_G, 1, E), lambda b: (b, 0, 0)),
        ],
        out_shape=[
            jax.ShapeDtypeStruct((B, T, C), jnp.float32),
            jax.ShapeDtypeStruct((B, 1, E), jnp.float32),
        ],
        compiler_params=pltpu.CompilerParams(
            dimension_semantics=("parallel",),
        ),
    )(x_flat, w_gate, W1, b1, W2, b2)

    loss = pl.pallas_call(
        _loss_body,
        out_shape=jax.ShapeDtypeStruct((1, 1), jnp.float32),
    )(gates)

    return y_flat.reshape(B, H, W, C), loss[0, 0]
